# baseline (speedup 1.0000x reference)
import functools

import jax
import jax.numpy as jnp
from jax import lax
from jax.experimental import pallas as pl
from jax.experimental.pallas import tpu as pltpu
from jax.experimental.pallas import tpu_sc as plsc

E = 16
N = 16384 * 2
NW = 32
CHUNK = N // NW
VECS = CHUNK // 16
ROWS = CHUNK // 128

_mesh = plsc.VectorSubcoreMesh(core_axis_name="c", subcore_axis_name="s")


def _worker_id():
    return lax.axis_index("c") * 16 + lax.axis_index("s")


@functools.partial(
    pl.kernel,
    mesh=_mesh,
    compiler_params=pltpu.CompilerParams(needs_layout_passes=False),
    out_type=jax.ShapeDtypeStruct((NW, E), jnp.int32),
    scratch_types=[
        pltpu.VMEM((CHUNK,), jnp.int32),
        pltpu.VMEM((E,), jnp.int32),
    ],
)
def _hist_kernel(eids_hbm, hist_hbm, ev, hv):
    w = _worker_id()
    pltpu.sync_copy(eids_hbm.at[pl.ds(w * CHUNK, CHUNK)], ev)
    iota = lax.iota(jnp.int32, 16)
    histv = jnp.zeros((16,), jnp.int32)
    for e in range(E):
        def body(i, acc, e=e):
            v = ev[pl.ds(pl.multiple_of(i * 16, 16), 16)]
            return acc + (v == e).astype(jnp.int32)
        acc = lax.fori_loop(0, VECS, body, jnp.zeros((16,), jnp.int32))
        histv = histv + jnp.where(iota == e, jnp.sum(acc), 0)
    hv[...] = histv
    pltpu.sync_copy(hv, hist_hbm.at[w])


@functools.partial(
    pl.kernel,
    mesh=_mesh,
    compiler_params=pltpu.CompilerParams(needs_layout_passes=False),
    out_type=(
        jax.ShapeDtypeStruct((N,), jnp.float32),
        jax.ShapeDtypeStruct((N,), jnp.int32),
        jax.ShapeDtypeStruct((E,), jnp.float32),
    ),
    scratch_types=[
        pltpu.VMEM((CHUNK,), jnp.int32),
        pltpu.VMEM((NW, E), jnp.int32),
        pltpu.VMEM((ROWS, 128), jnp.int32),
        pltpu.VMEM((ROWS, 128), jnp.int32),
        pltpu.VMEM((ROWS, 128), jnp.float32),
        pltpu.VMEM((E,), jnp.float32),
        pltpu.SemaphoreType.DMA,
    ],
)
def _sort_kernel(eids_hbm, scores_hbm, hist_hbm, sc_out, idx_out, cnt_out,
                 ev, histv, loc_idx, loc_dst, loc_sc, cnt_v, sem):
    w = _worker_id()
    base_elem = w * CHUNK
    pltpu.sync_copy(eids_hbm.at[pl.ds(base_elem, CHUNK)], ev)
    pltpu.sync_copy(hist_hbm, histv)

    col = jnp.zeros((16,), jnp.int32)
    pre = jnp.zeros((16,), jnp.int32)
    own = jnp.zeros((16,), jnp.int32)
    for r in range(NW):
        row = histv[r, :]
        col = col + row
        pre = pre + jnp.where(r < w, row, 0)
        own = own + jnp.where(r == w, row, 0)
    base_e = plsc.cumsum(col) - col
    offs = base_e + pre
    lstart = plsc.cumsum(own) - own
    delta = offs - lstart

    @pl.when(w == 0)
    def _():
        cnt_v[...] = col.astype(jnp.float32)
        pltpu.sync_copy(cnt_v, cnt_out)

    iota = lax.iota(jnp.int32, 16)
    for e in range(E):
        d_e = delta[e]
        def body(i, pos, e=e, d_e=d_e):
            off = pl.multiple_of(i * 16, 16)
            v = ev[pl.ds(off, 16)]
            m = v == e
            mi = m.astype(jnp.int32)
            incl = plsc.cumsum(mi)
            ldst = pos + incl - 1
            tok = iota + (base_elem + off)
            i0 = ldst >> 7
            i1 = ldst & 127
            plsc.store_scatter(loc_idx, [i0, i1], tok, mask=m)
            plsc.store_scatter(loc_dst, [i0, i1], d_e + ldst, mask=m)
            return pos + jnp.sum(mi)
        lax.fori_loop(0, VECS, body, lstart[e])

    gathers = [
        pltpu.async_copy(scores_hbm.at[loc_idx.at[j]], loc_sc.at[j], sem)
        for j in range(ROWS)
    ]
    for c in gathers:
        c.wait()
    scats = []
    for j in range(ROWS):
        scats.append(pltpu.async_copy(loc_idx.at[j], idx_out.at[loc_dst.at[j]], sem))
        scats.append(pltpu.async_copy(loc_sc.at[j], sc_out.at[loc_dst.at[j]], sem))
    for c in scats:
        c.wait()


@jax.jit
def _token_reorder(top_scores, selected_experts_indices):
    eids = selected_experts_indices.reshape(-1)
    scores = top_scores.reshape(-1)
    hist = _hist_kernel(eids)
    return _sort_kernel(eids, scores, hist)


def kernel(top_scores, selected_experts_indices):
    return _token_reorder(top_scores, selected_experts_indices)


# speedup vs baseline: 1.0443x; 1.0443x over previous
import functools

import jax
import jax.numpy as jnp
from jax import lax
from jax.experimental import pallas as pl
from jax.experimental.pallas import tpu as pltpu
from jax.experimental.pallas import tpu_sc as plsc

E = 16
N = 16384 * 2
NW = 32
CHUNK = N // NW
SEG = CHUNK // 16
ROWS = CHUNK // 128

_mesh = plsc.VectorSubcoreMesh(core_axis_name="c", subcore_axis_name="s")
_params = pltpu.CompilerParams(needs_layout_passes=False)


def _worker_id():
    return lax.axis_index("c") * 16 + lax.axis_index("s")


@functools.partial(
    pl.kernel,
    mesh=_mesh,
    compiler_params=_params,
    out_type=(
        jax.ShapeDtypeStruct((NW, E), jnp.int32),
        jax.ShapeDtypeStruct((NW * 16, E), jnp.int32),
    ),
    scratch_types=[
        pltpu.VMEM((CHUNK,), jnp.int32),
        pltpu.VMEM((16, E), jnp.int32),
        pltpu.VMEM((E,), jnp.int32),
    ],
)
def _hist_kernel(eids_hbm, blk_hbm, lane_hbm, ev, h2d, bs):
    w = _worker_id()
    pltpu.sync_copy(eids_hbm.at[pl.ds(w * CHUNK, CHUNK)], ev)
    zeros = jnp.zeros((16,), jnp.int32)
    for r in range(16):
        h2d[r, :] = zeros
    iota = lax.iota(jnp.int32, 16)
    seg = iota * SEG
    ones = jnp.ones((16,), jnp.int32)

    def body(t, c):
        v = plsc.load_gather(ev, [seg + t])
        plsc.addupdate_scatter(h2d, [iota, v], ones)
        return c

    lax.fori_loop(0, SEG, body, 0)
    acc = zeros
    for r in range(16):
        acc = acc + h2d[r, :]
    bs[...] = acc
    pltpu.sync_copy(h2d, lane_hbm.at[pl.ds(w * 16, 16)])
    pltpu.sync_copy(bs, blk_hbm.at[w])


@functools.partial(
    pl.kernel,
    mesh=_mesh,
    compiler_params=_params,
    out_type=(
        jax.ShapeDtypeStruct((N,), jnp.float32),
        jax.ShapeDtypeStruct((N,), jnp.int32),
        jax.ShapeDtypeStruct((E,), jnp.float32),
    ),
    scratch_types=[
        pltpu.VMEM((CHUNK,), jnp.int32),
        pltpu.VMEM((NW, E), jnp.int32),
        pltpu.VMEM((16, E), jnp.int32),
        pltpu.VMEM((16, E), jnp.int32),
        pltpu.VMEM((16, E), jnp.int32),
        pltpu.VMEM((ROWS, 128), jnp.int32),
        pltpu.VMEM((ROWS, 128), jnp.int32),
        pltpu.VMEM((ROWS, 128), jnp.float32),
        pltpu.VMEM((E,), jnp.float32),
        pltpu.SemaphoreType.DMA,
    ],
)
def _sort_kernel(eids_hbm, scores_hbm, blk_hbm, lane_hbm,
                 sc_out, idx_out, cnt_out,
                 ev, hb, hl, g2d, d2d, loc_idx, loc_dst, loc_sc, cnt_v, sem):
    w = _worker_id()
    base_elem = w * CHUNK
    pltpu.sync_copy(eids_hbm.at[pl.ds(base_elem, CHUNK)], ev)
    pltpu.sync_copy(blk_hbm, hb)
    pltpu.sync_copy(lane_hbm.at[pl.ds(w * 16, 16)], hl)

    col = jnp.zeros((16,), jnp.int32)
    pre = jnp.zeros((16,), jnp.int32)
    own = jnp.zeros((16,), jnp.int32)
    for r in range(NW):
        row = hb[r, :]
        col = col + row
        pre = pre + jnp.where(r < w, row, 0)
        own = own + jnp.where(r == w, row, 0)
    base_e = plsc.cumsum(col) - col
    rg = base_e + pre
    rl = plsc.cumsum(own) - own
    for lane in range(16):
        g2d[lane, :] = rg
        d2d[lane, :] = rg - rl
        hrow = hl[lane, :]
        rg = rg + hrow
        rl = rl + hrow

    @pl.when(w == 0)
    def _():
        cnt_v[...] = col.astype(jnp.float32)
        pltpu.sync_copy(cnt_v, cnt_out)

    iota = lax.iota(jnp.int32, 16)
    seg = iota * SEG
    ones = jnp.ones((16,), jnp.int32)

    def body(t, c):
        idxs = seg + t
        v = plsc.load_gather(ev, [idxs])
        gdst = plsc.load_gather(g2d, [iota, v])
        dl = plsc.load_gather(d2d, [iota, v])
        ldst = gdst - dl
        tok = base_elem + idxs
        i0 = ldst >> 7
        i1 = ldst & 127
        plsc.store_scatter(loc_idx, [i0, i1], tok)
        plsc.store_scatter(loc_dst, [i0, i1], gdst)
        plsc.addupdate_scatter(g2d, [iota, v], ones)
        return c

    lax.fori_loop(0, SEG, body, 0)

    gathers = [
        pltpu.async_copy(scores_hbm.at[loc_idx.at[j]], loc_sc.at[j], sem)
        for j in range(ROWS)
    ]
    for c in gathers:
        c.wait()
    scats = []
    for j in range(ROWS):
        scats.append(pltpu.async_copy(loc_idx.at[j], idx_out.at[loc_dst.at[j]], sem))
        scats.append(pltpu.async_copy(loc_sc.at[j], sc_out.at[loc_dst.at[j]], sem))
    for c in scats:
        c.wait()


@jax.jit
def _token_reorder(top_scores, selected_experts_indices):
    eids = selected_experts_indices.reshape(-1)
    scores = top_scores.reshape(-1)
    blk, lane = _hist_kernel(eids)
    return _sort_kernel(eids, scores, blk, lane)


def kernel(top_scores, selected_experts_indices):
    return _token_reorder(top_scores, selected_experts_indices)


# speedup vs baseline: 4.3293x; 4.1455x over previous
import functools

import jax
import jax.numpy as jnp
from jax import lax
from jax.experimental import pallas as pl
from jax.experimental.pallas import tpu as pltpu
from jax.experimental.pallas import tpu_sc as plsc

E = 16
N = 16384 * 2
NSUB = 16
CH = N // NSUB
SEG = CH // 16
NV = NSUB * 16
ROWS = CH // 128

_mesh = plsc.VectorSubcoreMesh(core_axis_name="c", subcore_axis_name="s")
_params = pltpu.CompilerParams(needs_layout_passes=False)


@functools.partial(
    pl.kernel,
    mesh=_mesh,
    compiler_params=_params,
    out_type=(
        jax.ShapeDtypeStruct((N,), jnp.float32),
        jax.ShapeDtypeStruct((N,), jnp.int32),
        jax.ShapeDtypeStruct((E,), jnp.float32),
    ),
    scratch_types=[
        pltpu.VMEM((CH,), jnp.int32),
        pltpu.VMEM((CH,), jnp.float32),
        pltpu.VMEM((16, E), jnp.int32),
        pltpu.VMEM((NV, E), jnp.int32),
        pltpu.VMEM((16, E), jnp.int32),
        pltpu.VMEM((16, E), jnp.int32),
        pltpu.VMEM((ROWS, 128), jnp.int32),
        pltpu.VMEM((ROWS, 128), jnp.int32),
        pltpu.VMEM((ROWS, 128), jnp.float32),
        pltpu.VMEM((E,), jnp.float32),
        pltpu.VMEM_SHARED((NV, E), jnp.int32),
        pltpu.VMEM_SHARED((N,), jnp.int32),
        pltpu.VMEM_SHARED((N,), jnp.float32),
        pltpu.SemaphoreType.DMA,
    ],
)
def _reorder_kernel(eids_hbm, scores_hbm, sc_out, idx_out, cnt_out,
                    ev, sv, h2d, hfull, g2d, d2d, loc_idx, loc_dst, loc_sc,
                    cnt_v, sh_hist, sh_idx, sh_sc, sem):
    c = lax.axis_index("c")
    s = lax.axis_index("s")
    base_elem = s * CH
    pltpu.sync_copy(eids_hbm.at[pl.ds(base_elem, CH)], ev)
    pltpu.sync_copy(scores_hbm.at[pl.ds(base_elem, CH)], sv)

    iota = lax.iota(jnp.int32, 16)
    zeros = jnp.zeros((16,), jnp.int32)
    ones = jnp.ones((16,), jnp.int32)
    for r in range(16):
        h2d[r, :] = zeros
    seg = iota * SEG

    def hbody(t, cc):
        v = plsc.load_gather(ev, [seg + t])
        plsc.addupdate_scatter(h2d, [iota, v], ones)
        return cc

    lax.fori_loop(0, SEG, hbody, 0)
    pltpu.sync_copy(h2d, sh_hist.at[pl.ds(s * 16, 16)])
    plsc.subcore_barrier()

    pltpu.sync_copy(sh_hist, hfull)
    lo = s * 16

    def obody(r, carry):
        col, pre, own = carry
        row = hfull[r, :]
        col = col + row
        pre = pre + jnp.where(r < lo, row, 0)
        own = own + jnp.where((r >= lo) & (r < lo + 16), row, 0)
        return col, pre, own

    col, pre, own = lax.fori_loop(0, NV, obody, (zeros, zeros, zeros))
    base_e = plsc.cumsum(col) - col
    rg = base_e + pre
    rl = plsc.cumsum(own) - own
    for lane in range(16):
        g2d[lane, :] = rg
        d2d[lane, :] = rg - rl
        hrow = hfull[lo + lane, :]
        rg = rg + hrow
        rl = rl + hrow

    @pl.when((c == 0) & (s == 0))
    def _():
        cnt_v[...] = col.astype(jnp.float32)
        pltpu.sync_copy(cnt_v, cnt_out)

    def body(t, cc):
        idxs = seg + t
        v = plsc.load_gather(ev, [idxs])
        gdst = plsc.load_gather(g2d, [iota, v])
        dl = plsc.load_gather(d2d, [iota, v])
        ldst = gdst - dl
        sc = plsc.load_gather(sv, [idxs])
        i0 = ldst >> 7
        i1 = ldst & 127
        plsc.store_scatter(loc_idx, [i0, i1], base_elem + idxs)
        plsc.store_scatter(loc_dst, [i0, i1], gdst)
        plsc.store_scatter(loc_sc, [i0, i1], sc)
        plsc.addupdate_scatter(g2d, [iota, v], ones)
        return cc

    lax.fori_loop(0, SEG, body, 0)

    scats = []
    for j in range(ROWS):
        scats.append(pltpu.async_copy(loc_idx.at[j], sh_idx.at[loc_dst.at[j]], sem))
        scats.append(pltpu.async_copy(loc_sc.at[j], sh_sc.at[loc_dst.at[j]], sem))
    for cp in scats:
        cp.wait()
    plsc.subcore_barrier()

    @pl.when(c == 0)
    def _():
        pltpu.sync_copy(sh_idx.at[pl.ds(base_elem, CH)], ev)
        pltpu.sync_copy(ev, idx_out.at[pl.ds(base_elem, CH)])

    @pl.when(c == 1)
    def _():
        pltpu.sync_copy(sh_sc.at[pl.ds(base_elem, CH)], sv)
        pltpu.sync_copy(sv, sc_out.at[pl.ds(base_elem, CH)])


@jax.jit
def _token_reorder(top_scores, selected_experts_indices):
    eids = selected_experts_indices.reshape(-1)
    scores = top_scores.reshape(-1)
    return _reorder_kernel(eids, scores)


def kernel(top_scores, selected_experts_indices):
    return _token_reorder(top_scores, selected_experts_indices)


# speedup vs baseline: 4.3317x; 1.0005x over previous
import functools

import jax
import jax.numpy as jnp
from jax import lax
from jax.experimental import pallas as pl
from jax.experimental.pallas import tpu as pltpu
from jax.experimental.pallas import tpu_sc as plsc

E = 16
N = 16384 * 2
NSUB = 16
CH = N // NSUB
SEG = CH // 16
NV = NSUB * 16
ROWS = CH // 128

_mesh = plsc.VectorSubcoreMesh(core_axis_name="c", subcore_axis_name="s")
_params = pltpu.CompilerParams(needs_layout_passes=False)


@functools.partial(
    pl.kernel,
    mesh=_mesh,
    compiler_params=_params,
    out_type=(
        jax.ShapeDtypeStruct((N,), jnp.float32),
        jax.ShapeDtypeStruct((N,), jnp.int32),
        jax.ShapeDtypeStruct((E,), jnp.float32),
    ),
    scratch_types=[
        pltpu.VMEM((CH // 128, 128), jnp.int32),
        pltpu.VMEM((CH // 128, 128), jnp.float32),
        pltpu.VMEM((16, E), jnp.int32),
        pltpu.VMEM((NV, E), jnp.int32),
        pltpu.VMEM((16, E), jnp.int32),
        pltpu.VMEM((16, E), jnp.int32),
        pltpu.VMEM((ROWS, 128), jnp.int32),
        pltpu.VMEM((ROWS, 128), jnp.int32),
        pltpu.VMEM((ROWS, 128), jnp.float32),
        pltpu.VMEM((E,), jnp.float32),
        pltpu.VMEM((CH,), jnp.int32),
        pltpu.VMEM((CH,), jnp.float32),
        pltpu.VMEM_SHARED((NV, E), jnp.int32),
        pltpu.VMEM_SHARED((N,), jnp.int32),
        pltpu.VMEM_SHARED((N,), jnp.float32),
        pltpu.SemaphoreType.DMA,
    ],
)
def _reorder_kernel(eids_hbm, scores_hbm, sc_out, idx_out, cnt_out,
                    ev, sv, h2d, hfull, g2d, d2d, loc_idx, loc_dst, loc_sc,
                    cnt_v, bi, bs, sh_hist, sh_idx, sh_sc, sem):
    c = lax.axis_index("c")
    s = lax.axis_index("s")
    base_elem = s * CH
    base_row = s * (CH // 128)
    pltpu.sync_copy(eids_hbm.at[pl.ds(base_row, CH // 128), :], ev)
    pltpu.sync_copy(scores_hbm.at[pl.ds(base_row, CH // 128), :], sv)

    iota = lax.iota(jnp.int32, 16)
    zeros = jnp.zeros((16,), jnp.int32)
    ones = jnp.ones((16,), jnp.int32)
    for r in range(16):
        h2d[r, :] = zeros
    seg = iota * SEG

    def hbody(t, cc):
        idxs = seg + t
        v = plsc.load_gather(ev, [idxs >> 7, idxs & 127])
        plsc.addupdate_scatter(h2d, [iota, v], ones)
        return cc

    lax.fori_loop(0, SEG, hbody, 0)
    pltpu.sync_copy(h2d, sh_hist.at[pl.ds(s * 16, 16)])
    plsc.subcore_barrier()

    pltpu.sync_copy(sh_hist, hfull)
    lo = s * 16

    def obody(r, carry):
        col, pre, own = carry
        row = hfull[r, :]
        col = col + row
        pre = pre + jnp.where(r < lo, row, 0)
        own = own + jnp.where((r >= lo) & (r < lo + 16), row, 0)
        return col, pre, own

    col, pre, own = lax.fori_loop(0, NV, obody, (zeros, zeros, zeros))
    base_e = plsc.cumsum(col) - col
    rg = base_e + pre
    rl = plsc.cumsum(own) - own
    for lane in range(16):
        g2d[lane, :] = rg
        d2d[lane, :] = rg - rl
        hrow = hfull[lo + lane, :]
        rg = rg + hrow
        rl = rl + hrow

    @pl.when((c == 0) & (s == 0))
    def _():
        cnt_v[...] = col.astype(jnp.float32)
        pltpu.sync_copy(cnt_v, cnt_out)

    def body(t, cc):
        idxs = seg + t
        v = plsc.load_gather(ev, [idxs >> 7, idxs & 127])
        gdst = plsc.load_gather(g2d, [iota, v])
        dl = plsc.load_gather(d2d, [iota, v])
        ldst = gdst - dl
        sc = plsc.load_gather(sv, [idxs >> 7, idxs & 127])
        i0 = ldst >> 7
        i1 = ldst & 127
        plsc.store_scatter(loc_idx, [i0, i1], base_elem + idxs)
        plsc.store_scatter(loc_dst, [i0, i1], gdst)
        plsc.store_scatter(loc_sc, [i0, i1], sc)
        plsc.addupdate_scatter(g2d, [iota, v], ones)
        return cc

    lax.fori_loop(0, SEG, body, 0)

    scats = []
    for j in range(ROWS):
        scats.append(pltpu.async_copy(loc_idx.at[j], sh_idx.at[loc_dst.at[j]], sem))
        scats.append(pltpu.async_copy(loc_sc.at[j], sh_sc.at[loc_dst.at[j]], sem))
    for cp in scats:
        cp.wait()
    plsc.subcore_barrier()

    @pl.when(c == 0)
    def _():
        pltpu.sync_copy(sh_idx.at[pl.ds(base_elem, CH)], bi)
        pltpu.sync_copy(bi, idx_out.at[pl.ds(base_elem, CH)])

    @pl.when(c == 1)
    def _():
        pltpu.sync_copy(sh_sc.at[pl.ds(base_elem, CH)], bs)
        pltpu.sync_copy(bs, sc_out.at[pl.ds(base_elem, CH)])


@jax.jit
def _token_reorder(top_scores, selected_experts_indices):
    eids = selected_experts_indices.reshape(N // 128, 128)
    scores = top_scores.reshape(N // 128, 128)
    return _reorder_kernel(eids, scores)


def kernel(top_scores, selected_experts_indices):
    return _token_reorder(top_scores, selected_experts_indices)


# speedup vs baseline: 4.5209x; 1.0437x over previous
import functools

import jax
import jax.numpy as jnp
from jax import lax
from jax.experimental import pallas as pl
from jax.experimental.pallas import tpu as pltpu
from jax.experimental.pallas import tpu_sc as plsc

E = 16
N = 16384 * 2
NSUB = 16
CH = N // NSUB
SEG = CH // 16
NV = NSUB * 16
ROWS = CH // 128

_mesh = plsc.VectorSubcoreMesh(core_axis_name="c", subcore_axis_name="s")
_params = pltpu.CompilerParams(needs_layout_passes=False)


@functools.partial(
    pl.kernel,
    mesh=_mesh,
    compiler_params=_params,
    out_type=(
        jax.ShapeDtypeStruct((N,), jnp.float32),
        jax.ShapeDtypeStruct((N,), jnp.int32),
        jax.ShapeDtypeStruct((E,), jnp.float32),
    ),
    scratch_types=[
        pltpu.VMEM((CH // 128, 128), jnp.int32),
        pltpu.VMEM((CH // 128, 128), jnp.float32),
        pltpu.VMEM((16, E), jnp.int32),
        pltpu.VMEM((E,), jnp.int32),
        pltpu.VMEM((NSUB, E), jnp.int32),
        pltpu.VMEM((16, E), jnp.int32),
        pltpu.VMEM((16, E), jnp.int32),
        pltpu.VMEM((ROWS, 128), jnp.int32),
        pltpu.VMEM((ROWS, 128), jnp.int32),
        pltpu.VMEM((ROWS, 128), jnp.float32),
        pltpu.VMEM((E,), jnp.float32),
        pltpu.VMEM((CH,), jnp.int32),
        pltpu.VMEM((CH,), jnp.float32),
        pltpu.VMEM_SHARED((NSUB, E), jnp.int32),
        pltpu.VMEM_SHARED((N,), jnp.int32),
        pltpu.VMEM_SHARED((N,), jnp.float32),
        pltpu.SemaphoreType.DMA,
    ],
)
def _reorder_kernel(eids_hbm, scores_hbm, sc_out, idx_out, cnt_out,
                    ev, sv, h2d, bsv, blkv, g2d, d2d, loc_idx, loc_dst, loc_sc,
                    cnt_v, bi, bs, sh_blk, sh_idx, sh_sc, sem):
    c = lax.axis_index("c")
    s = lax.axis_index("s")
    base_elem = s * CH
    base_row = s * (CH // 128)
    pltpu.sync_copy(eids_hbm.at[pl.ds(base_row, CH // 128), :], ev)
    pltpu.sync_copy(scores_hbm.at[pl.ds(base_row, CH // 128), :], sv)

    iota = lax.iota(jnp.int32, 16)
    zeros = jnp.zeros((16,), jnp.int32)
    ones = jnp.ones((16,), jnp.int32)
    for r in range(16):
        h2d[r, :] = zeros
    seg = iota * SEG

    def hbody(t, cc):
        for u in range(2):
            idxs = seg + (t * 2 + u)
            v = plsc.load_gather(ev, [idxs >> 7, idxs & 127])
            plsc.addupdate_scatter(h2d, [iota, v], ones)
        return cc

    lax.fori_loop(0, SEG // 2, hbody, 0)
    bsum = jnp.zeros((16,), jnp.int32)
    for r in range(16):
        bsum = bsum + h2d[r, :]
    bsv[...] = bsum
    pltpu.sync_copy(bsv, sh_blk.at[s])
    plsc.subcore_barrier()

    pltpu.sync_copy(sh_blk, blkv)
    col = jnp.zeros((16,), jnp.int32)
    pre = jnp.zeros((16,), jnp.int32)
    for r in range(NSUB):
        row = blkv[r, :]
        col = col + row
        pre = pre + jnp.where(r < s, row, 0)
    base_e = plsc.cumsum(col) - col
    rg = base_e + pre
    rl = plsc.cumsum(bsum) - bsum
    for lane in range(16):
        g2d[lane, :] = rg
        d2d[lane, :] = rg - rl
        hrow = h2d[lane, :]
        rg = rg + hrow
        rl = rl + hrow

    @pl.when((c == 0) & (s == 0))
    def _():
        cnt_v[...] = col.astype(jnp.float32)
        pltpu.sync_copy(cnt_v, cnt_out)

    def body(t, cc):
        for u in range(2):
            idxs = seg + (t * 2 + u)
            v = plsc.load_gather(ev, [idxs >> 7, idxs & 127])
            gdst = plsc.load_gather(g2d, [iota, v])
            dl = plsc.load_gather(d2d, [iota, v])
            ldst = gdst - dl
            sc = plsc.load_gather(sv, [idxs >> 7, idxs & 127])
            i0 = ldst >> 7
            i1 = ldst & 127
            plsc.store_scatter(loc_idx, [i0, i1], base_elem + idxs)
            plsc.store_scatter(loc_dst, [i0, i1], gdst)
            plsc.store_scatter(loc_sc, [i0, i1], sc)
            plsc.addupdate_scatter(g2d, [iota, v], ones)
        return cc

    lax.fori_loop(0, SEG // 2, body, 0)

    scats = []
    for j in range(ROWS):
        scats.append(pltpu.async_copy(loc_idx.at[j], sh_idx.at[loc_dst.at[j]], sem))
        scats.append(pltpu.async_copy(loc_sc.at[j], sh_sc.at[loc_dst.at[j]], sem))
    for cp in scats:
        cp.wait()
    plsc.subcore_barrier()

    @pl.when(c == 0)
    def _():
        pltpu.sync_copy(sh_idx.at[pl.ds(base_elem, CH)], bi)
        pltpu.sync_copy(bi, idx_out.at[pl.ds(base_elem, CH)])

    @pl.when(c == 1)
    def _():
        pltpu.sync_copy(sh_sc.at[pl.ds(base_elem, CH)], bs)
        pltpu.sync_copy(bs, sc_out.at[pl.ds(base_elem, CH)])


@jax.jit
def _token_reorder(top_scores, selected_experts_indices):
    eids = selected_experts_indices.reshape(N // 128, 128)
    scores = top_scores.reshape(N // 128, 128)
    return _reorder_kernel(eids, scores)


def kernel(top_scores, selected_experts_indices):
    return _token_reorder(top_scores, selected_experts_indices)
